# Initial kernel scaffold; baseline (speedup 1.0000x reference)
import jax
import jax.numpy as jnp
from jax.experimental import pallas as pl


def kernel(input, weight, locations):
    raise NotImplementedError("write your pallas kernel here")


# speedup vs baseline: 7.4394x; 7.4394x over previous
import jax
import jax.numpy as jnp
from jax.experimental import pallas as pl
from jax.experimental.pallas import tpu as pltpu

B = 512
D = 128
K = 1024


def _som_body(x_ref, w_ref, loc_ref, locs_out, idx_out, loss_out):
    x = x_ref[:]
    v = w_ref[:] - 1e-6
    dots = jax.lax.dot_general(
        x, v, (((1,), (0,)), ((), ())),
        preferred_element_type=jnp.float32,
        precision=jax.lax.Precision.HIGHEST,
    )
    xsq = jnp.sum(x * x, axis=1, keepdims=True)
    vsq = jnp.sum(v * v, axis=0, keepdims=True)
    d2 = jnp.maximum(xsq + vsq - 2.0 * dots, 0.0)
    mind2 = jnp.min(d2, axis=1, keepdims=True)
    iota = jax.lax.broadcasted_iota(jnp.int32, (B, K), 1)
    idx = jnp.min(jnp.where(d2 == mind2, iota, K), axis=1, keepdims=True)
    idx_out[:] = idx
    onehot = (iota == idx).astype(jnp.float32)
    locs_out[:] = jax.lax.dot_general(
        onehot, loc_ref[:], (((1,), (0,)), ((), ())),
        preferred_element_type=jnp.float32,
        precision=jax.lax.Precision.HIGHEST,
    )
    loss_out[0, 0] = jnp.sum(jnp.sqrt(mind2)) / B


def kernel(input, weight, locations):
    locs, idx, loss = pl.pallas_call(
        _som_body,
        out_shape=(
            jax.ShapeDtypeStruct((B, 2), jnp.float32),
            jax.ShapeDtypeStruct((B, 1), jnp.int32),
            jax.ShapeDtypeStruct((1, 1), jnp.float32),
        ),
        out_specs=(
            pl.BlockSpec(memory_space=pltpu.VMEM),
            pl.BlockSpec(memory_space=pltpu.VMEM),
            pl.BlockSpec(memory_space=pltpu.SMEM),
        ),
    )(input, weight, locations)
    return locs.reshape(B, 1, 2), loss.reshape(()), idx
